# baseline (device time: 87720 ns/iter reference)
import jax
import jax.numpy as jnp
from jax import lax
from jax.experimental import pallas as pl
from jax.experimental.pallas import tpu as pltpu

N_Z = 4
N_Q = 4
H, Dh, Dr = 16, 128, 32
HQ = H // N_Q
DC = 128
SCALE = (Dh + Dr) ** -0.5
F32 = jnp.float32
BF16 = jnp.bfloat16


def _mla_fused(x3, Wdkv, Wuk, Wuv, Wq, Wqr, WkrT, Wo):
    _, s, d = x3.shape
    dq = d // N_Q
    n_hops = N_Z - 1
    qchunk = dq // N_Z

    def body(x_ref, wdkv_ref, wuk_ref, wuv_ref, wq_ref, wqr_ref, wkrt_ref,
             wo_ref, out_ref,
             cbuf, wukbuf, wuvbuf, wqstage, wqrstage,
             qs, kacc, vacc, qrq, krs, oq, oall, wo2,
             send_sems, recv_sems, wq_sem,
             osend_sems, orecv_sems, wo_sems, xybar):
        mx = lax.axis_index("x")
        my = lax.axis_index("y")
        mz = lax.axis_index("z")
        hq = mx * 2 + my
        left = (mz + N_Z - 1) % N_Z
        right = (mz + 1) % N_Z

        wq_copy = pltpu.make_async_copy(
            wq_ref.at[:, pl.ds(hq * dq, dq)], wqstage, wq_sem)
        wq_copy.start()
        wo_own = pltpu.make_async_copy(
            wo_ref.at[pl.ds(hq * dq, dq), :], wo2.at[0], wo_sems.at[0])
        wo_own.start()

        peers = []
        for i in (3, 1, 2):
            p_hq = hq ^ i
            peers.append((p_hq, p_hq // 2, p_hq % 2))
        for p_hq, px, py in peers:
            pl.semaphore_signal(
                xybar, inc=1,
                device_id=(px, py, mz),
                device_id_type=pl.DeviceIdType.MESH,
            )

        for qq in range(N_Q):
            @pl.when(hq == qq)
            def _():
                wukbuf[n_hops] = wuk_ref[:, qq * dq:(qq + 1) * dq].astype(BF16)
                wuvbuf[n_hops] = wuv_ref[:, qq * dq:(qq + 1) * dq].astype(BF16)
                wqrstage[...] = wqr_ref[:, qq * HQ * Dr:(qq + 1) * HQ * Dr]
        xv = x_ref[0]
        cbuf[n_hops] = jnp.dot(
            xv, wdkv_ref[...], preferred_element_type=F32).astype(BF16)

        barrier = pltpu.get_barrier_semaphore()
        for nbr in (left, right):
            pl.semaphore_signal(
                barrier, inc=1,
                device_id=(mx, my, nbr),
                device_id_type=pl.DeviceIdType.MESH,
            )
        pl.semaphore_wait(barrier, 2)

        def start_hop(h, src_slot):
            rdmas = []
            for t, buf in enumerate((cbuf, wukbuf, wuvbuf)):
                rdma = pltpu.make_async_remote_copy(
                    src_ref=buf.at[src_slot],
                    dst_ref=buf.at[h],
                    send_sem=send_sems.at[h, t],
                    recv_sem=recv_sems.at[h, t],
                    device_id=(mx, my, right),
                    device_id_type=pl.DeviceIdType.MESH,
                )
                rdma.start()
                rdmas.append(rdma)
            return rdmas

        def fold(slot, first=False):
            cv = cbuf[slot]
            kp = jnp.dot(cv, wukbuf[slot], preferred_element_type=F32)
            vp = jnp.dot(cv, wuvbuf[slot], preferred_element_type=F32)
            if first:
                kacc[...] = kp
                vacc[...] = vp
            else:
                kacc[...] += kp
                vacc[...] += vp

        rdmas = start_hop(0, n_hops)
        fold(n_hops, first=True)
        qrq[...] = jnp.dot(
            xv, wqrstage[...], preferred_element_type=F32) * SCALE
        krs[...] = lax.dot_general(
            xv, wkrt_ref[...], (((1,), (1,)), ((), ())),
            preferred_element_type=F32)
        wq_copy.wait()
        qs[:, 0:qchunk] = jnp.dot(
            xv, wqstage[:, 0:qchunk], preferred_element_type=F32) * SCALE
        for r in rdmas:
            r.wait()

        for h in range(1, n_hops):
            rdmas = start_hop(h, h - 1)
            fold(h - 1)
            qs[:, h * qchunk:(h + 1) * qchunk] = jnp.dot(
                xv, wqstage[:, h * qchunk:(h + 1) * qchunk],
                preferred_element_type=F32) * SCALE
            for r in rdmas:
                r.wait()

        fold(n_hops - 1)
        qs[:, n_hops * qchunk:] = jnp.dot(
            xv, wqstage[:, n_hops * qchunk:],
            preferred_element_type=F32) * SCALE

        pl.semaphore_wait(xybar, 3)

        kr_v = krs[...]
        sends = []
        for j in range(HQ):
            q_h = qs[:, j * Dh:(j + 1) * Dh]
            k_h = kacc[:, j * Dh:(j + 1) * Dh]
            qr_h = qrq[:, j * Dr:(j + 1) * Dr]
            sc = lax.dot_general(
                q_h, k_h, (((1,), (1,)), ((), ())),
                preferred_element_type=F32,
            )
            sc += lax.dot_general(
                qr_h, kr_v, (((1,), (1,)), ((), ())),
                preferred_element_type=F32,
            )
            p = jnp.exp(sc)
            denom = jnp.sum(p, axis=1, keepdims=True)
            o_un = jnp.dot(
                p, vacc[:, j * Dh:(j + 1) * Dh], preferred_element_type=F32)
            oq[:, j * Dh:(j + 1) * Dh] = (o_un / denom).astype(BF16)
            for i, (p_hq, px, py) in enumerate(peers):
                rdma = pltpu.make_async_remote_copy(
                    src_ref=oq.at[:, pl.ds(j * Dh, Dh)],
                    dst_ref=oall.at[hq, :, pl.ds(j * Dh, Dh)],
                    send_sem=osend_sems.at[i, j],
                    recv_sem=orecv_sems.at[i, j],
                    device_id=(px, py, mz),
                    device_id_type=pl.DeviceIdType.MESH,
                )
                rdma.start()
                sends.append(rdma)

        wo_y = pltpu.make_async_copy(
            wo_ref.at[pl.ds(peers[1][0] * dq, dq), :], wo2.at[1],
            wo_sems.at[1])
        wo_y.start()

        wo_own.wait()
        out_ref[0] = lax.dot_general(
            oq[...], wo2[0],
            (((1,), (0,)), ((), ())), preferred_element_type=F32)
        wo_x = pltpu.make_async_copy(
            wo_ref.at[pl.ds(peers[2][0] * dq, dq), :], wo2.at[0],
            wo_sems.at[2])
        wo_x.start()

        def wait_peer_head(i, p_hq, px, py, j):
            recv = pltpu.make_async_remote_copy(
                src_ref=oq.at[:, pl.ds(j * Dh, Dh)],
                dst_ref=oall.at[p_hq, :, pl.ds(j * Dh, Dh)],
                send_sem=osend_sems.at[i, j],
                recv_sem=orecv_sems.at[i, j],
                device_id=(px, py, mz),
                device_id_type=pl.DeviceIdType.MESH,
            )
            recv.wait_recv()

        p_hq, px, py = peers[1]
        for j in range(HQ):
            wait_peer_head(1, p_hq, px, py, j)
        wo_y.wait()
        out_ref[0] += lax.dot_general(
            oall[p_hq], wo2[1],
            (((1,), (0,)), ((), ())), preferred_element_type=F32)
        wo_diag = pltpu.make_async_copy(
            wo_ref.at[pl.ds(peers[0][0] * dq, dq), :], wo2.at[1],
            wo_sems.at[3])
        wo_diag.start()

        p_hq, px, py = peers[2]
        for j in range(HQ):
            wait_peer_head(2, p_hq, px, py, j)
        wo_x.wait()
        out_ref[0] += lax.dot_general(
            oall[p_hq], wo2[0],
            (((1,), (0,)), ((), ())), preferred_element_type=F32)

        p_hq, px, py = peers[0]
        wo_diag.wait()
        for j in range(HQ):
            wait_peer_head(0, p_hq, px, py, j)
            out_ref[0] += lax.dot_general(
                oall[p_hq, :, j * Dh:(j + 1) * Dh],
                wo2[1, j * Dh:(j + 1) * Dh, :],
                (((1,), (0,)), ((), ())), preferred_element_type=F32)
        for rdma in sends:
            rdma.wait_send()

    vm = pl.BlockSpec(memory_space=pltpu.VMEM)
    hbm = pl.BlockSpec(memory_space=pl.ANY)
    return pl.pallas_call(
        body,
        in_specs=[vm, vm, vm, vm, hbm, vm, vm, hbm],
        out_shape=jax.ShapeDtypeStruct((1, s, d), F32),
        scratch_shapes=[
            pltpu.VMEM((N_Z, s, DC), BF16),
            pltpu.VMEM((N_Z, DC, dq), BF16),
            pltpu.VMEM((N_Z, DC, dq), BF16),
            pltpu.VMEM((d, dq), F32),
            pltpu.VMEM((d, HQ * Dr), F32),
            pltpu.VMEM((s, dq), F32),
            pltpu.VMEM((s, dq), F32),
            pltpu.VMEM((s, dq), F32),
            pltpu.VMEM((s, HQ * Dr), F32),
            pltpu.VMEM((s, Dr), F32),
            pltpu.VMEM((s, dq), BF16),
            pltpu.VMEM((N_Q, s, dq), BF16),
            pltpu.VMEM((2, dq, d), F32),
            pltpu.SemaphoreType.DMA((n_hops, 3)),
            pltpu.SemaphoreType.DMA((n_hops, 3)),
            pltpu.SemaphoreType.DMA,
            pltpu.SemaphoreType.DMA((3, HQ)),
            pltpu.SemaphoreType.DMA((3, HQ)),
            pltpu.SemaphoreType.DMA((4,)),
            pltpu.SemaphoreType.REGULAR,
        ],
        compiler_params=pltpu.CompilerParams(
            collective_id=0, vmem_limit_bytes=63 * 1024 * 1024),
    )(x3, Wdkv, Wuk, Wuv, Wq, Wqr, WkrT, Wo)


def kernel(x, Wdkv, Wuk, Wuv, Wq, Wqr, Wkr, Wo):
    return _mla_fused(x, Wdkv, Wuk, Wuv, Wq, Wqr, Wkr.T, Wo)


# device time: 83183 ns/iter; 1.0545x vs baseline; 1.0545x over previous
import jax
import jax.numpy as jnp
from jax import lax
from jax.experimental import pallas as pl
from jax.experimental.pallas import tpu as pltpu

N_Z = 4
N_Q = 4
H, Dh, Dr = 16, 128, 32
HQ = H // N_Q
DC = 128
SCALE = (Dh + Dr) ** -0.5
F32 = jnp.float32
BF16 = jnp.bfloat16


def _mla_fused(x3, Wdkv, Wuk, Wuv, Wq, Wqr, WkrT, Wo):
    _, s, d = x3.shape
    dq = d // N_Q
    n_hops = N_Z - 1
    qchunk = dq // N_Z

    def body(x_ref, wdkv_ref, wuk_ref, wuv_ref, wq_ref, wqr_ref, wkrt_ref,
             wo_ref, out_ref,
             cbuf, wukbuf, wuvbuf, wqstage, wqrstage,
             qs, kacc, vacc, qrq, krs, oq, oall, wo2,
             send_sems, recv_sems, wq_sem,
             osend_sems, orecv_sems, wo_sems, xybar):
        mx = lax.axis_index("x")
        my = lax.axis_index("y")
        mz = lax.axis_index("z")
        hq = mx * 2 + my
        left = (mz + N_Z - 1) % N_Z
        right = (mz + 1) % N_Z

        wq_copy = pltpu.make_async_copy(
            wq_ref.at[:, pl.ds(hq * dq, dq)], wqstage, wq_sem)
        wq_copy.start()
        wo_own = pltpu.make_async_copy(
            wo_ref.at[pl.ds(hq * dq, dq), :], wo2.at[0], wo_sems.at[0])
        wo_own.start()

        peers = []
        for i in (1, 2, 3):
            p_hq = hq ^ i
            peers.append((p_hq, p_hq // 2, p_hq % 2))
        for p_hq, px, py in peers:
            pl.semaphore_signal(
                xybar, inc=1,
                device_id=(px, py, mz),
                device_id_type=pl.DeviceIdType.MESH,
            )

        for qq in range(N_Q):
            @pl.when(hq == qq)
            def _():
                wukbuf[n_hops] = wuk_ref[:, qq * dq:(qq + 1) * dq].astype(BF16)
                wuvbuf[n_hops] = wuv_ref[:, qq * dq:(qq + 1) * dq].astype(BF16)
                wqrstage[...] = wqr_ref[:, qq * HQ * Dr:(qq + 1) * HQ * Dr]
        xv = x_ref[0]
        cbuf[n_hops] = jnp.dot(
            xv, wdkv_ref[...], preferred_element_type=F32).astype(BF16)

        barrier = pltpu.get_barrier_semaphore()
        for nbr in (left, right):
            pl.semaphore_signal(
                barrier, inc=1,
                device_id=(mx, my, nbr),
                device_id_type=pl.DeviceIdType.MESH,
            )
        pl.semaphore_wait(barrier, 2)

        def start_hop(h, src_slot):
            rdmas = []
            for t, buf in enumerate((cbuf, wukbuf, wuvbuf)):
                rdma = pltpu.make_async_remote_copy(
                    src_ref=buf.at[src_slot],
                    dst_ref=buf.at[h],
                    send_sem=send_sems.at[h, t],
                    recv_sem=recv_sems.at[h, t],
                    device_id=(mx, my, right),
                    device_id_type=pl.DeviceIdType.MESH,
                )
                rdma.start()
                rdmas.append(rdma)
            return rdmas

        def fold(slot, first=False):
            cv = cbuf[slot]
            kp = jnp.dot(cv, wukbuf[slot], preferred_element_type=F32)
            vp = jnp.dot(cv, wuvbuf[slot], preferred_element_type=F32)
            if first:
                kacc[...] = kp
                vacc[...] = vp
            else:
                kacc[...] += kp
                vacc[...] += vp

        rdmas = start_hop(0, n_hops)
        fold(n_hops, first=True)
        qrq[...] = jnp.dot(
            xv, wqrstage[...], preferred_element_type=F32) * SCALE
        krs[...] = lax.dot_general(
            xv, wkrt_ref[...], (((1,), (1,)), ((), ())),
            preferred_element_type=F32)
        wq_copy.wait()
        qs[:, 0:qchunk] = jnp.dot(
            xv, wqstage[:, 0:qchunk], preferred_element_type=F32) * SCALE
        for r in rdmas:
            r.wait()

        for h in range(1, n_hops):
            rdmas = start_hop(h, h - 1)
            fold(h - 1)
            qs[:, h * qchunk:(h + 1) * qchunk] = jnp.dot(
                xv, wqstage[:, h * qchunk:(h + 1) * qchunk],
                preferred_element_type=F32) * SCALE
            for r in rdmas:
                r.wait()

        fold(n_hops - 1)
        qs[:, n_hops * qchunk:] = jnp.dot(
            xv, wqstage[:, n_hops * qchunk:],
            preferred_element_type=F32) * SCALE

        pl.semaphore_wait(xybar, 3)

        kr_v = krs[...]
        sends = []
        for j in range(HQ):
            q_h = qs[:, j * Dh:(j + 1) * Dh]
            k_h = kacc[:, j * Dh:(j + 1) * Dh]
            qr_h = qrq[:, j * Dr:(j + 1) * Dr]
            sc = lax.dot_general(
                q_h, k_h, (((1,), (1,)), ((), ())),
                preferred_element_type=F32,
            )
            sc += lax.dot_general(
                qr_h, kr_v, (((1,), (1,)), ((), ())),
                preferred_element_type=F32,
            )
            p = jnp.exp(sc)
            denom = jnp.sum(p, axis=1, keepdims=True)
            o_un = jnp.dot(
                p, vacc[:, j * Dh:(j + 1) * Dh], preferred_element_type=F32)
            oq[:, j * Dh:(j + 1) * Dh] = (o_un / denom).astype(BF16)
            for i, (p_hq, px, py) in enumerate(peers):
                rdma = pltpu.make_async_remote_copy(
                    src_ref=oq.at[:, pl.ds(j * Dh, Dh)],
                    dst_ref=oall.at[hq, :, pl.ds(j * Dh, Dh)],
                    send_sem=osend_sems.at[i, j],
                    recv_sem=orecv_sems.at[i, j],
                    device_id=(px, py, mz),
                    device_id_type=pl.DeviceIdType.MESH,
                )
                rdma.start()
                sends.append(rdma)

        wo_p1 = pltpu.make_async_copy(
            wo_ref.at[pl.ds(peers[0][0] * dq, dq), :], wo2.at[1],
            wo_sems.at[1])
        wo_p1.start()

        wo_own.wait()
        out_ref[0] = lax.dot_general(
            oq[...], wo2[0],
            (((1,), (0,)), ((), ())), preferred_element_type=F32)

        wo_next = wo_p1
        for i, (p_hq, px, py) in enumerate(peers):
            if i + 1 < len(peers):
                wo_after = pltpu.make_async_copy(
                    wo_ref.at[pl.ds(peers[i + 1][0] * dq, dq), :],
                    wo2.at[i % 2], wo_sems.at[i + 2])
                wo_after.start()
            for j in range(HQ):
                recv = pltpu.make_async_remote_copy(
                    src_ref=oq.at[:, pl.ds(j * Dh, Dh)],
                    dst_ref=oall.at[p_hq, :, pl.ds(j * Dh, Dh)],
                    send_sem=osend_sems.at[i, j],
                    recv_sem=orecv_sems.at[i, j],
                    device_id=(px, py, mz),
                    device_id_type=pl.DeviceIdType.MESH,
                )
                recv.wait_recv()
            wo_next.wait()
            out_ref[0] += lax.dot_general(
                oall[p_hq], wo2[(i + 1) % 2],
                (((1,), (0,)), ((), ())), preferred_element_type=F32)
            if i + 1 < len(peers):
                wo_next = wo_after
        for rdma in sends:
            rdma.wait_send()

    vm = pl.BlockSpec(memory_space=pltpu.VMEM)
    hbm = pl.BlockSpec(memory_space=pl.ANY)
    return pl.pallas_call(
        body,
        in_specs=[vm, vm, vm, vm, hbm, vm, vm, hbm],
        out_shape=jax.ShapeDtypeStruct((1, s, d), F32),
        scratch_shapes=[
            pltpu.VMEM((N_Z, s, DC), BF16),
            pltpu.VMEM((N_Z, DC, dq), BF16),
            pltpu.VMEM((N_Z, DC, dq), BF16),
            pltpu.VMEM((d, dq), F32),
            pltpu.VMEM((d, HQ * Dr), F32),
            pltpu.VMEM((s, dq), F32),
            pltpu.VMEM((s, dq), F32),
            pltpu.VMEM((s, dq), F32),
            pltpu.VMEM((s, HQ * Dr), F32),
            pltpu.VMEM((s, Dr), F32),
            pltpu.VMEM((s, dq), BF16),
            pltpu.VMEM((N_Q, s, dq), BF16),
            pltpu.VMEM((2, dq, d), F32),
            pltpu.SemaphoreType.DMA((n_hops, 3)),
            pltpu.SemaphoreType.DMA((n_hops, 3)),
            pltpu.SemaphoreType.DMA,
            pltpu.SemaphoreType.DMA((3, HQ)),
            pltpu.SemaphoreType.DMA((3, HQ)),
            pltpu.SemaphoreType.DMA((4,)),
            pltpu.SemaphoreType.REGULAR,
        ],
        compiler_params=pltpu.CompilerParams(
            collective_id=0, vmem_limit_bytes=63 * 1024 * 1024),
    )(x3, Wdkv, Wuk, Wuv, Wq, Wqr, WkrT, Wo)


def kernel(x, Wdkv, Wuk, Wuv, Wq, Wqr, Wkr, Wo):
    return _mla_fused(x, Wdkv, Wuk, Wuv, Wq, Wqr, Wkr.T, Wo)
